# Initial kernel scaffold; baseline (speedup 1.0000x reference)
import jax
import jax.numpy as jnp
from jax.experimental import pallas as pl


def kernel(edge_index, embedding):
    raise NotImplementedError("write your pallas kernel here")


# speedup vs baseline: 5.9739x; 5.9739x over previous
import jax
import jax.numpy as jnp
from jax import lax
from jax.experimental import pallas as pl
from jax.experimental.pallas import tpu as pltpu
from jax.experimental.pallas import tpu_sc as plsc

N_NODES = 10000
N_EDGES = 320000
DIM = 128
N_LAYERS = 3

NC = 2
NS = 16
L = 16
HALF = DIM // NC
NP = 10240
TN = NP // NS
EC = 128
G = 8
NG = 20
CHUNKS = G * NG
EPT = CHUNKS * EC
EPAD = NS * EPT
NCH = TN // EC

_F32 = jnp.float32
_I32 = jnp.int32


def _newton_rsqrt(x):
    y = 1.0 / x
    for _ in range(26):
        y = y * (1.5 - 0.5 * x * y * y)
    return y


def _body(src_hbm, dst_hbm, emb_hbm, out_hbm, h_hbm, hs_hbm,
          out_t, agg, dgo, dgi, ibs, ibd, gbuf, nbuf, zbuf, norms,
          onesv, zvec):
    c = lax.axis_index("c")
    s = lax.axis_index("s")
    nbase = s * TN
    hbase = c * NP + nbase
    coff = c * NP
    z16 = jnp.zeros((L,), _F32)

    def _zrow(i, _):
        for k in range(HALF // L):
            zbuf[i, pl.ds(k * L, L)] = z16
        return 0
    lax.fori_loop(0, EC, _zrow, 0)

    def _zvec(i, _):
        zvec[pl.ds(i * L, L)] = z16
        return 0
    lax.fori_loop(0, TN // L, _zvec, 0)

    def _ones(i, _):
        onesv[pl.ds(i * L, L)] = jnp.ones((L,), _F32)
        return 0
    lax.fori_loop(0, EC // L, _ones, 0)

    pltpu.sync_copy(zvec, dgo.at[pl.ds(nbase, TN)])
    pltpu.sync_copy(zvec, dgi.at[pl.ds(nbase, TN)])
    for cb in range(NCH):
        pltpu.sync_copy(zbuf, agg.at[pl.ds(nbase + cb * EC, EC)])
    plsc.subcore_barrier()

    def _deg(g, _):
        pltpu.sync_copy(src_hbm.at[s, pl.ds(g * G, G)], ibs.at[0])
        pltpu.sync_copy(dst_hbm.at[s, pl.ds(g * G, G)], ibd.at[0])
        for jj in range(G):
            pltpu.sync_copy(onesv, dgo.at[ibs.at[0, jj]], add=True)
            pltpu.sync_copy(onesv, dgi.at[ibd.at[0, jj]], add=True)
        return 0
    lax.fori_loop(0, NG, _deg, 0)
    plsc.subcore_barrier()

    pltpu.sync_copy(dgi.at[pl.ds(nbase, TN)], norms.at[0])
    pltpu.sync_copy(dgo.at[pl.ds(nbase, TN)], norms.at[1])

    def _norm(i, _):
        for d in range(2):
            sl = pl.ds(i * L, L)
            x = jnp.maximum(norms[d, sl], 1.0)
            norms[d, sl] = _newton_rsqrt(x)
        return 0
    lax.fori_loop(0, TN // L, _norm, 0)

    def node_phase(layer):
        for cb in range(NCH):
            base = nbase + cb * EC
            if layer == 0:
                pltpu.sync_copy(emb_hbm.at[pl.ds(coff + base, EC)], nbuf)
            else:
                pltpu.sync_copy(agg.at[pl.ds(base, EC)], nbuf)
                pltpu.sync_copy(zbuf, agg.at[pl.ds(base, EC)])

            def _rows(g, _):
                ndv = norms[0, pl.ds(cb * EC + g * L, L)]
                nsv = norms[1, pl.ds(cb * EC + g * L, L)]
                for t in range(L):
                    i = g * L + t
                    nd = ndv[t]
                    ns = nsv[t]
                    for k in range(HALF // L):
                        sl = pl.ds(k * L, L)
                        v = nbuf[i, sl]
                        if layer == 0:
                            out_t[cb * EC + i, sl] = v
                            nbuf[i, sl] = v * ns
                        elif layer < N_LAYERS:
                            out_t[cb * EC + i, sl] = out_t[cb * EC + i, sl] + v * nd
                            nbuf[i, sl] = v * (nd * ns)
                        else:
                            h = v * nd
                            out_t[cb * EC + i, sl] = out_t[cb * EC + i, sl] + h
                            nbuf[i, sl] = h
                return 0
            lax.fori_loop(0, EC // L, _rows, 0)

            if layer < N_LAYERS:
                pltpu.sync_copy(nbuf, hs_hbm.at[pl.ds(coff + base, EC)])
            else:
                pltpu.sync_copy(nbuf, h_hbm.at[pl.ds(coff + base, EC)])
        if layer == N_LAYERS:
            pltpu.sync_copy(out_t, out_hbm.at[pl.ds(hbase, TN)])

    def edge_phase():
        def _grp(g, _):
            pltpu.sync_copy(src_hbm.at[s, pl.ds(g * G, G)], ibs.at[0])
            pltpu.sync_copy(dst_hbm.at[s, pl.ds(g * G, G)], ibd.at[0])
            for jj in range(G):
                for k in range(EC // L):
                    sl = pl.ds(k * L, L)
                    ibs[0, jj, sl] = ibs[0, jj, sl] + coff
            for jj in range(G):
                pltpu.sync_copy(hs_hbm.at[ibs.at[0, jj]], gbuf.at[0])
                pltpu.sync_copy(gbuf.at[0], agg.at[ibd.at[0, jj]], add=True)
            return 0
        lax.fori_loop(0, NG, _grp, 0)

    node_phase(0)
    plsc.subcore_barrier()
    for layer in range(1, N_LAYERS + 1):
        edge_phase()
        plsc.subcore_barrier()
        node_phase(layer)
        if layer < N_LAYERS:
            plsc.subcore_barrier()


_lightgcn_sc = pl.kernel(
    _body,
    out_type=(
        jax.ShapeDtypeStruct((NC * NP, HALF), _F32),
        jax.ShapeDtypeStruct((NC * NP, HALF), _F32),
        jax.ShapeDtypeStruct((NC * NP, HALF), _F32),
    ),
    mesh=plsc.VectorSubcoreMesh(core_axis_name="c", subcore_axis_name="s"),
    compiler_params=pltpu.CompilerParams(use_tc_tiling_on_sc=False),
    scratch_types=[
        pltpu.VMEM((TN, HALF), _F32),
        pltpu.VMEM_SHARED((NP, HALF), _F32),
        pltpu.VMEM_SHARED((NP,), _F32),
        pltpu.VMEM_SHARED((NP,), _F32),
        pltpu.VMEM((2, G, EC), _I32),
        pltpu.VMEM((2, G, EC), _I32),
        pltpu.VMEM((2, EC, HALF), _F32),
        pltpu.VMEM((EC, HALF), _F32),
        pltpu.VMEM((EC, HALF), _F32),
        pltpu.VMEM((2, TN), _F32),
        pltpu.VMEM((EC,), _F32),
        pltpu.VMEM((TN,), _F32),
    ],
)


@jax.jit
def kernel(edge_index, embedding):
    src = edge_index[0].astype(_I32)
    dst = edge_index[1].astype(_I32)
    pad_e = EPAD - N_EDGES
    fill = jnp.full((pad_e,), N_NODES, _I32)
    srcp = jnp.concatenate([src, fill]).reshape(NS, CHUNKS, EC)
    dstp = jnp.concatenate([dst, fill]).reshape(NS, CHUNKS, EC)
    embp = jnp.pad(embedding, ((0, NP - N_NODES), (0, 0)))
    emb_r = embp.reshape(NP, NC, HALF).transpose(1, 0, 2).reshape(NC * NP, HALF)

    out_r, h_r, _ = _lightgcn_sc(srcp, dstp, emb_r)

    def _unsplit(a):
        return (a.reshape(NC, NP, HALF).transpose(1, 0, 2)
                .reshape(NP, DIM)[:N_NODES])

    return (_unsplit(out_r), _unsplit(h_r))


# speedup vs baseline: 6.4055x; 1.0722x over previous
import jax
import jax.numpy as jnp
from jax import lax
from jax.experimental import pallas as pl
from jax.experimental.pallas import tpu as pltpu
from jax.experimental.pallas import tpu_sc as plsc

N_NODES = 10000
N_EDGES = 320000
DIM = 128
N_LAYERS = 3

NC = 2
NS = 16
L = 16
HALF = DIM // NC
NP = 10240
TN = NP // NS
EC = 128
G = 8
NG = 20
CHUNKS = G * NG
EPT = CHUNKS * EC
EPAD = NS * EPT
NCH = TN // EC

_F32 = jnp.float32
_I32 = jnp.int32


def _newton_rsqrt(x):
    y = 1.0 / x
    for _ in range(26):
        y = y * (1.5 - 0.5 * x * y * y)
    return y


def _body(src_hbm, dst_hbm, emb_hbm, out_hbm, h_hbm, hs_hbm,
          out_t, agg, dgo, dgi, ibs, ibd, gbuf, nbuf, norms, onesv, zvec,
          gsem0, gsem1, gsem2, ssem0, ssem1, ssem2,
          isems0, isems1, isemd0, isemd1, wsem0, wsem1, zsem):
    c = lax.axis_index("c")
    s = lax.axis_index("s")
    nbase = s * TN
    hbase = c * NP + nbase
    coff = c * NP
    z16 = jnp.zeros((L,), _F32)
    gsem = (gsem0, gsem1, gsem2)
    ssem = (ssem0, ssem1, ssem2)
    isems = (isems0, isems1)
    isemd = (isemd0, isemd1)
    wsem = (wsem0, wsem1)

    def _zvec(i, _):
        zvec[pl.ds(i * L, L)] = z16
        return 0
    lax.fori_loop(0, TN // L, _zvec, 0)

    def _ones(i, _):
        onesv[pl.ds(i * L, L)] = jnp.ones((L,), _F32)
        return 0
    lax.fori_loop(0, EC // L, _ones, 0)

    def _zero_gbuf0(i, _):
        for k in range(HALF // L):
            gbuf[0, i, pl.ds(k * L, L)] = z16
        return 0

    pltpu.sync_copy(zvec, dgo.at[pl.ds(nbase, TN)])
    pltpu.sync_copy(zvec, dgi.at[pl.ds(nbase, TN)])
    lax.fori_loop(0, EC, _zero_gbuf0, 0)
    for cb in range(NCH):
        pltpu.sync_copy(gbuf.at[0], agg.at[pl.ds(nbase + cb * EC, EC)])
    plsc.subcore_barrier()

    def _load_idx(plane, g, half):
        ds = pltpu.async_copy(src_hbm.at[plane, s, pl.ds(g * G, G)],
                              ibs.at[half], isems[half])
        dd = pltpu.async_copy(dst_hbm.at[s, pl.ds(g * G, G)],
                              ibd.at[half], isemd[half])
        return ds, dd

    def _wait_idx(half):
        pltpu.make_async_copy(src_hbm.at[0, s, pl.ds(0, G)],
                              ibs.at[half], isems[half]).wait()
        pltpu.make_async_copy(dst_hbm.at[s, pl.ds(0, G)],
                              ibd.at[half], isemd[half]).wait()

    def _run_groups(plane, process_group):
        _load_idx(plane, 0, 0)
        _load_idx(plane, 1, 1)

        def _pair(m, _):
            for half in range(2):
                g = 2 * m + half
                _wait_idx(half)
                process_group(half)
                gn = lax.rem(g + 2, NG)
                _load_idx(plane, gn, half)
            return 0
        lax.fori_loop(0, NG // 2, _pair, 0)
        _wait_idx(0)
        _wait_idx(1)

    def _deg_group(half):
        descs = []
        for jj in range(G):
            descs.append(pltpu.async_copy(
                onesv, dgo.at[ibs.at[half, jj]], gsem[0], add=True))
            descs.append(pltpu.async_copy(
                onesv, dgi.at[ibd.at[half, jj]], gsem[1], add=True))
        for d in descs:
            d.wait()

    _run_groups(0, _deg_group)
    plsc.subcore_barrier()

    pltpu.sync_copy(dgi.at[pl.ds(nbase, TN)], norms.at[0])
    pltpu.sync_copy(dgo.at[pl.ds(nbase, TN)], norms.at[1])

    def _norm(i, _):
        for d in range(2):
            sl = pl.ds(i * L, L)
            x = jnp.maximum(norms[d, sl], 1.0)
            norms[d, sl] = _newton_rsqrt(x)
        return 0
    lax.fori_loop(0, TN // L, _norm, 0)

    def node_phase(layer):
        if layer > 0:
            lax.fori_loop(0, EC, _zero_gbuf0, 0)
        wdescs = {}
        zdescs = []
        for cb in range(NCH):
            base = nbase + cb * EC
            use_g = cb % 2
            if cb >= 2:
                wdescs[cb - 2].wait()
            buf = gbuf.at[1] if use_g else nbuf
            if layer == 0:
                pltpu.sync_copy(emb_hbm.at[pl.ds(coff + base, EC)], buf)
            else:
                pltpu.sync_copy(agg.at[pl.ds(base, EC)], buf)
                zdescs.append(pltpu.async_copy(
                    gbuf.at[0], agg.at[pl.ds(base, EC)], zsem))

            def _rows(g, _):
                ndv = norms[0, pl.ds(cb * EC + g * L, L)]
                nsv = norms[1, pl.ds(cb * EC + g * L, L)]
                for t in range(L):
                    i = g * L + t
                    nd = ndv[t]
                    ns = nsv[t]
                    for k in range(HALF // L):
                        sl = pl.ds(k * L, L)
                        v = gbuf[1, i, sl] if use_g else nbuf[i, sl]
                        if layer == 0:
                            out_t[cb * EC + i, sl] = v
                            w = v * ns
                        elif layer < N_LAYERS:
                            out_t[cb * EC + i, sl] = out_t[cb * EC + i, sl] + v * nd
                            w = v * (nd * ns)
                        else:
                            w = v * nd
                            out_t[cb * EC + i, sl] = out_t[cb * EC + i, sl] + w
                        if use_g:
                            gbuf[1, i, sl] = w
                        else:
                            nbuf[i, sl] = w
                return 0
            lax.fori_loop(0, EC // L, _rows, 0)

            tgt = hs_hbm if layer < N_LAYERS else h_hbm
            wdescs[cb] = pltpu.async_copy(
                buf, tgt.at[pl.ds(coff + base, EC)], wsem[cb % 2])
        wdescs[NCH - 2].wait()
        wdescs[NCH - 1].wait()
        for d in zdescs:
            d.wait()
        if layer == N_LAYERS:
            pltpu.sync_copy(out_t, out_hbm.at[pl.ds(hbase, TN)])

    def _edge_group(half):
        dg = {}
        sc = {}

        def _scatter(jj):
            b = jj % 3
            dg[jj].wait()
            sc[jj] = pltpu.async_copy(
                gbuf.at[b], agg.at[ibd.at[half, jj]], ssem[b], add=True)

        for jj in range(G):
            b = jj % 3
            if jj >= 3:
                sc[jj - 3].wait()
            dg[jj] = pltpu.async_copy(
                hs_hbm.at[ibs.at[half, jj]], gbuf.at[b], gsem[b])
            if jj >= 1:
                _scatter(jj - 1)
        _scatter(G - 1)
        for jj in (G - 3, G - 2, G - 1):
            sc[jj].wait()

    node_phase(0)
    plsc.subcore_barrier()
    for layer in range(1, N_LAYERS + 1):
        _run_groups(c, _edge_group)
        plsc.subcore_barrier()
        node_phase(layer)
        if layer < N_LAYERS:
            plsc.subcore_barrier()


_lightgcn_sc = pl.kernel(
    _body,
    out_type=(
        jax.ShapeDtypeStruct((NC * NP, HALF), _F32),
        jax.ShapeDtypeStruct((NC * NP, HALF), _F32),
        jax.ShapeDtypeStruct((NC * NP, HALF), _F32),
    ),
    mesh=plsc.VectorSubcoreMesh(core_axis_name="c", subcore_axis_name="s"),
    compiler_params=pltpu.CompilerParams(use_tc_tiling_on_sc=False),
    scratch_types=[
        pltpu.VMEM((TN, HALF), _F32),
        pltpu.VMEM_SHARED((NP, HALF), _F32),
        pltpu.VMEM_SHARED((NP,), _F32),
        pltpu.VMEM_SHARED((NP,), _F32),
        pltpu.VMEM((2, G, EC), _I32),
        pltpu.VMEM((2, G, EC), _I32),
        pltpu.VMEM((3, EC, HALF), _F32),
        pltpu.VMEM((EC, HALF), _F32),
        pltpu.VMEM((2, TN), _F32),
        pltpu.VMEM((EC,), _F32),
        pltpu.VMEM((TN,), _F32),
    ] + [pltpu.SemaphoreType.DMA] * 13,
)


@jax.jit
def kernel(edge_index, embedding):
    src = edge_index[0].astype(_I32)
    dst = edge_index[1].astype(_I32)
    pad_e = EPAD - N_EDGES
    fill = jnp.full((pad_e,), N_NODES, _I32)
    srcp = jnp.concatenate([src, fill]).reshape(NS, CHUNKS, EC)
    dstp = jnp.concatenate([dst, fill]).reshape(NS, CHUNKS, EC)
    src2 = jnp.stack([srcp, srcp + NP])
    embp = jnp.pad(embedding, ((0, NP - N_NODES), (0, 0)))
    emb_r = embp.reshape(NP, NC, HALF).transpose(1, 0, 2).reshape(NC * NP, HALF)

    out_r, h_r, _ = _lightgcn_sc(src2, dstp, emb_r)

    def _unsplit(a):
        return (a.reshape(NC, NP, HALF).transpose(1, 0, 2)
                .reshape(NP, DIM)[:N_NODES])

    return (_unsplit(out_r), _unsplit(h_r))


# speedup vs baseline: 6.9390x; 1.0833x over previous
import jax
import jax.numpy as jnp
from jax import lax
from jax.experimental import pallas as pl
from jax.experimental.pallas import tpu as pltpu
from jax.experimental.pallas import tpu_sc as plsc

N_NODES = 10000
N_EDGES = 320000
DIM = 128
N_LAYERS = 3

NC = 2
NS = 16
L = 16
HALF = DIM // NC
NP = 10240
TN = NP // NS
NB = 128
EB = 256
G = 4
NG = 20
CHUNKS = G * NG
EPT = CHUNKS * EB
EPAD = NS * EPT
NCH = TN // NB

_F32 = jnp.float32
_I32 = jnp.int32


def _newton_rsqrt(x):
    y = 1.0 / x
    for _ in range(26):
        y = y * (1.5 - 0.5 * x * y * y)
    return y


def _body(src_hbm, dst_hbm, emb_hbm, out_hbm, h_hbm, hs_hbm,
          out_t, agg, dgo, dgi, ibs, ibd, gbuf, nbuf, norms, onesv, zvec,
          gsem0, gsem1, ssem0, ssem1,
          isems0, isems1, isemd0, isemd1, wsem0, wsem1, zsem):
    c = lax.axis_index("c")
    s = lax.axis_index("s")
    nbase = s * TN
    hbase = c * NP + nbase
    coff = c * NP
    z16 = jnp.zeros((L,), _F32)
    gsem = (gsem0, gsem1)
    ssem = (ssem0, ssem1)
    isems = (isems0, isems1)
    isemd = (isemd0, isemd1)
    wsem = (wsem0, wsem1)

    def _zvec(i, _):
        zvec[pl.ds(i * L, L)] = z16
        return 0
    lax.fori_loop(0, TN // L, _zvec, 0)

    def _ones(i, _):
        onesv[pl.ds(i * L, L)] = jnp.ones((L,), _F32)
        return 0
    lax.fori_loop(0, EB // L, _ones, 0)

    def _zero_gbuf0(i, _):
        for k in range(HALF // L):
            gbuf[0, i, pl.ds(k * L, L)] = z16
        return 0

    pltpu.sync_copy(zvec, dgo.at[pl.ds(nbase, TN)])
    pltpu.sync_copy(zvec, dgi.at[pl.ds(nbase, TN)])
    lax.fori_loop(0, NB, _zero_gbuf0, 0)
    for cb in range(NCH):
        pltpu.sync_copy(gbuf.at[0, pl.ds(0, NB)],
                        agg.at[pl.ds(nbase + cb * NB, NB)])
    plsc.subcore_barrier()

    def _load_idx(plane, g, half):
        pltpu.async_copy(src_hbm.at[plane, s, pl.ds(g * G, G)],
                         ibs.at[half], isems[half])
        pltpu.async_copy(dst_hbm.at[s, pl.ds(g * G, G)],
                         ibd.at[half], isemd[half])

    def _wait_idx(half):
        pltpu.make_async_copy(src_hbm.at[0, s, pl.ds(0, G)],
                              ibs.at[half], isems[half]).wait()
        pltpu.make_async_copy(dst_hbm.at[s, pl.ds(0, G)],
                              ibd.at[half], isemd[half]).wait()

    def _run_groups(plane, process_group):
        _load_idx(plane, 0, 0)
        _load_idx(plane, 1, 1)

        def _pair(m, _):
            for half in range(2):
                g = 2 * m + half
                _wait_idx(half)
                process_group(half)
                gn = lax.rem(g + 2, NG)
                _load_idx(plane, gn, half)
            return 0
        lax.fori_loop(0, NG // 2, _pair, 0)
        _wait_idx(0)
        _wait_idx(1)

    def _deg_group(half):
        descs = []
        for jj in range(G):
            descs.append(pltpu.async_copy(
                onesv, dgo.at[ibs.at[half, jj]], gsem[0], add=True))
            descs.append(pltpu.async_copy(
                onesv, dgi.at[ibd.at[half, jj]], gsem[1], add=True))
        for d in descs:
            d.wait()

    _run_groups(0, _deg_group)
    plsc.subcore_barrier()

    pltpu.sync_copy(dgi.at[pl.ds(nbase, TN)], norms.at[0])
    pltpu.sync_copy(dgo.at[pl.ds(nbase, TN)], norms.at[1])

    def _norm(i, _):
        for d in range(2):
            sl = pl.ds(i * L, L)
            x = jnp.maximum(norms[d, sl], 1.0)
            norms[d, sl] = _newton_rsqrt(x)
        return 0
    lax.fori_loop(0, TN // L, _norm, 0)

    def node_phase(layer):
        if layer > 0:
            lax.fori_loop(0, NB, _zero_gbuf0, 0)
        wdescs = {}
        zdescs = []
        for cb in range(NCH):
            base = nbase + cb * NB
            use_g = cb % 2
            if cb >= 2:
                wdescs[cb - 2].wait()
            buf = gbuf.at[1, pl.ds(0, NB)] if use_g else nbuf
            if layer == 0:
                pltpu.sync_copy(emb_hbm.at[pl.ds(coff + base, NB)], buf)
            else:
                pltpu.sync_copy(agg.at[pl.ds(base, NB)], buf)
                zdescs.append(pltpu.async_copy(
                    gbuf.at[0, pl.ds(0, NB)], agg.at[pl.ds(base, NB)], zsem))

            def _rows(g, _):
                ndv = norms[0, pl.ds(cb * NB + g * L, L)]
                nsv = norms[1, pl.ds(cb * NB + g * L, L)]
                for t in range(L):
                    i = g * L + t
                    nd = ndv[t]
                    ns = nsv[t]
                    for k in range(HALF // L):
                        sl = pl.ds(k * L, L)
                        v = gbuf[1, i, sl] if use_g else nbuf[i, sl]
                        if layer == 0:
                            out_t[cb * NB + i, sl] = v
                            w = v * ns
                        elif layer < N_LAYERS:
                            out_t[cb * NB + i, sl] = out_t[cb * NB + i, sl] + v * nd
                            w = v * (nd * ns)
                        else:
                            w = v * nd
                            out_t[cb * NB + i, sl] = out_t[cb * NB + i, sl] + w
                        if use_g:
                            gbuf[1, i, sl] = w
                        else:
                            nbuf[i, sl] = w
                return 0
            lax.fori_loop(0, NB // L, _rows, 0)

            tgt = hs_hbm if layer < N_LAYERS else h_hbm
            wdescs[cb] = pltpu.async_copy(
                buf, tgt.at[pl.ds(coff + base, NB)], wsem[cb % 2])
        wdescs[NCH - 2].wait()
        wdescs[NCH - 1].wait()
        for d in zdescs:
            d.wait()
        if layer == N_LAYERS:
            pltpu.sync_copy(out_t, out_hbm.at[pl.ds(hbase, TN)])

    def _edge_group(half):
        dg = {}
        sc = {}

        def _scatter(jj):
            b = jj % 2
            dg[jj].wait()
            sc[jj] = pltpu.async_copy(
                gbuf.at[b], agg.at[ibd.at[half, jj]], ssem[b], add=True)

        for jj in range(G):
            b = jj % 2
            if jj >= 2:
                sc[jj - 2].wait()
            dg[jj] = pltpu.async_copy(
                hs_hbm.at[ibs.at[half, jj]], gbuf.at[b], gsem[b])
            if jj >= 1:
                _scatter(jj - 1)
        _scatter(G - 1)
        sc[G - 2].wait()
        sc[G - 1].wait()

    node_phase(0)
    plsc.subcore_barrier()
    for layer in range(1, N_LAYERS + 1):
        _run_groups(c, _edge_group)
        plsc.subcore_barrier()
        node_phase(layer)
        if layer < N_LAYERS:
            plsc.subcore_barrier()


_lightgcn_sc = pl.kernel(
    _body,
    out_type=(
        jax.ShapeDtypeStruct((NC * NP, HALF), _F32),
        jax.ShapeDtypeStruct((NC * NP, HALF), _F32),
        jax.ShapeDtypeStruct((NC * NP, HALF), _F32),
    ),
    mesh=plsc.VectorSubcoreMesh(core_axis_name="c", subcore_axis_name="s"),
    compiler_params=pltpu.CompilerParams(use_tc_tiling_on_sc=False),
    scratch_types=[
        pltpu.VMEM((TN, HALF), _F32),
        pltpu.VMEM_SHARED((NP, HALF), _F32),
        pltpu.VMEM_SHARED((NP,), _F32),
        pltpu.VMEM_SHARED((NP,), _F32),
        pltpu.VMEM((2, G, EB), _I32),
        pltpu.VMEM((2, G, EB), _I32),
        pltpu.VMEM((2, EB, HALF), _F32),
        pltpu.VMEM((NB, HALF), _F32),
        pltpu.VMEM((2, TN), _F32),
        pltpu.VMEM((EB,), _F32),
        pltpu.VMEM((TN,), _F32),
    ] + [pltpu.SemaphoreType.DMA] * 11,
)


@jax.jit
def kernel(edge_index, embedding):
    src = edge_index[0].astype(_I32)
    dst = edge_index[1].astype(_I32)
    pad_e = EPAD - N_EDGES
    fill = jnp.full((pad_e,), N_NODES, _I32)
    srcp = jnp.concatenate([src, fill]).reshape(NS, CHUNKS, EB)
    dstp = jnp.concatenate([dst, fill]).reshape(NS, CHUNKS, EB)
    src2 = jnp.stack([srcp, srcp + NP])
    embp = jnp.pad(embedding, ((0, NP - N_NODES), (0, 0)))
    emb_r = embp.reshape(NP, NC, HALF).transpose(1, 0, 2).reshape(NC * NP, HALF)

    out_r, h_r, _ = _lightgcn_sc(src2, dstp, emb_r)

    def _unsplit(a):
        return (a.reshape(NC, NP, HALF).transpose(1, 0, 2)
                .reshape(NP, DIM)[:N_NODES])

    return (_unsplit(out_r), _unsplit(h_r))


# speedup vs baseline: 11.3097x; 1.6299x over previous
import jax
import jax.numpy as jnp
from jax import lax
from jax.experimental import pallas as pl
from jax.experimental.pallas import tpu as pltpu
from jax.experimental.pallas import tpu_sc as plsc

N_NODES = 10000
N_EDGES = 320000
DIM = 128
N_LAYERS = 3

NC = 2
NS = 16
L = 16
HALF = DIM // NC
NP = 10240
TN = NP // NS
NB = 128
EB = 128
G = 8
NG = 20
CHUNKS = G * NG
EPT = CHUNKS * EB
EPAD = NS * EPT
NCH = TN // NB

_F32 = jnp.float32
_I32 = jnp.int32


def _newton_rsqrt(x):
    y = 1.0 / x
    for _ in range(26):
        y = y * (1.5 - 0.5 * x * y * y)
    return y


def _body(src_hbm, dst_hbm, emb_hbm, out_hbm, h_hbm,
          agg, hs_sp, dgo, dgi, ibs, ibd, gbuf, nbuf, obuf, norms,
          onesv, zvec,
          gsem0, gsem1, ssem0, ssem1,
          isems0, isems1, isemd0, isemd1, zsem):
    c = lax.axis_index("c")
    s = lax.axis_index("s")
    nbase = s * TN
    hbase = c * NP + nbase
    coff = c * NP
    z16 = jnp.zeros((L,), _F32)
    gsem = (gsem0, gsem1)
    ssem = (ssem0, ssem1)
    isems = (isems0, isems1)
    isemd = (isemd0, isemd1)

    def _zvec(i, _):
        zvec[pl.ds(i * L, L)] = z16
        return 0
    lax.fori_loop(0, TN // L, _zvec, 0)

    def _ones(i, _):
        onesv[pl.ds(i * L, L)] = jnp.ones((L,), _F32)
        return 0
    lax.fori_loop(0, EB // L, _ones, 0)

    def _zero_gbuf0(i, _):
        for k in range(HALF // L):
            gbuf[0, i, pl.ds(k * L, L)] = z16
        return 0

    pltpu.sync_copy(zvec, dgo.at[pl.ds(nbase, TN)])
    pltpu.sync_copy(zvec, dgi.at[pl.ds(nbase, TN)])
    lax.fori_loop(0, NB, _zero_gbuf0, 0)
    for cb in range(NCH):
        pltpu.sync_copy(gbuf.at[0], agg.at[pl.ds(nbase + cb * NB, NB)])
    plsc.subcore_barrier()

    def _load_idx(g, half):
        pltpu.async_copy(src_hbm.at[s, pl.ds(g * G, G)],
                         ibs.at[half], isems[half])
        pltpu.async_copy(dst_hbm.at[s, pl.ds(g * G, G)],
                         ibd.at[half], isemd[half])

    def _wait_idx(half):
        pltpu.make_async_copy(src_hbm.at[s, pl.ds(0, G)],
                              ibs.at[half], isems[half]).wait()
        pltpu.make_async_copy(dst_hbm.at[s, pl.ds(0, G)],
                              ibd.at[half], isemd[half]).wait()

    def _run_groups(process_group):
        _load_idx(0, 0)
        _load_idx(1, 1)

        def _pair(m, _):
            for half in range(2):
                g = 2 * m + half
                _wait_idx(half)
                process_group(half)
                gn = lax.rem(g + 2, NG)
                _load_idx(gn, half)
            return 0
        lax.fori_loop(0, NG // 2, _pair, 0)
        _wait_idx(0)
        _wait_idx(1)

    def _deg_group(half):
        descs = []
        for jj in range(G):
            descs.append(pltpu.async_copy(
                onesv, dgo.at[ibs.at[half, jj]], gsem[0], add=True))
            descs.append(pltpu.async_copy(
                onesv, dgi.at[ibd.at[half, jj]], gsem[1], add=True))
        for d in descs:
            d.wait()

    _run_groups(_deg_group)
    plsc.subcore_barrier()

    pltpu.sync_copy(dgi.at[pl.ds(nbase, TN)], norms.at[0])
    pltpu.sync_copy(dgo.at[pl.ds(nbase, TN)], norms.at[1])

    def _norm(i, _):
        for d in range(2):
            sl = pl.ds(i * L, L)
            x = jnp.maximum(norms[d, sl], 1.0)
            norms[d, sl] = _newton_rsqrt(x)
        return 0
    lax.fori_loop(0, TN // L, _norm, 0)

    def node_phase(layer):
        if layer > 0:
            lax.fori_loop(0, NB, _zero_gbuf0, 0)
        zdescs = []
        for cb in range(NCH):
            base = nbase + cb * NB
            if layer == 0:
                pltpu.sync_copy(emb_hbm.at[pl.ds(coff + base, NB)], nbuf)
                pltpu.sync_copy(nbuf, out_hbm.at[pl.ds(coff + base, NB)])
            else:
                pltpu.sync_copy(agg.at[pl.ds(base, NB)], nbuf)
                zdescs.append(pltpu.async_copy(
                    gbuf.at[0], agg.at[pl.ds(base, NB)], zsem))
                pltpu.sync_copy(out_hbm.at[pl.ds(coff + base, NB)], obuf)

            def _rows(g, _):
                ndv = norms[0, pl.ds(cb * NB + g * L, L)]
                nsv = norms[1, pl.ds(cb * NB + g * L, L)]
                for t in range(L):
                    i = g * L + t
                    nd = ndv[t]
                    ns = nsv[t]
                    for k in range(HALF // L):
                        sl = pl.ds(k * L, L)
                        v = nbuf[i, sl]
                        if layer == 0:
                            nbuf[i, sl] = v * ns
                        elif layer < N_LAYERS:
                            obuf[i, sl] = obuf[i, sl] + v * nd
                            nbuf[i, sl] = v * (nd * ns)
                        else:
                            w = v * nd
                            obuf[i, sl] = obuf[i, sl] + w
                            nbuf[i, sl] = w
                return 0
            lax.fori_loop(0, NB // L, _rows, 0)

            if layer > 0:
                pltpu.sync_copy(obuf, out_hbm.at[pl.ds(coff + base, NB)])
            if layer < N_LAYERS:
                pltpu.sync_copy(nbuf, hs_sp.at[pl.ds(base, NB)])
            else:
                pltpu.sync_copy(nbuf, h_hbm.at[pl.ds(coff + base, NB)])
        for d in zdescs:
            d.wait()

    def _edge_group(half):
        dg = {}
        sc = {}

        def _scatter(jj):
            b = jj % 2
            dg[jj].wait()
            sc[jj] = pltpu.async_copy(
                gbuf.at[b], agg.at[ibd.at[half, jj]], ssem[b], add=True)

        for jj in range(G):
            b = jj % 2
            if jj >= 2:
                sc[jj - 2].wait()
            dg[jj] = pltpu.async_copy(
                hs_sp.at[ibs.at[half, jj]], gbuf.at[b], gsem[b])
            if jj >= 1:
                _scatter(jj - 1)
        _scatter(G - 1)
        sc[G - 2].wait()
        sc[G - 1].wait()

    node_phase(0)
    plsc.subcore_barrier()
    for layer in range(1, N_LAYERS + 1):
        _run_groups(_edge_group)
        plsc.subcore_barrier()
        node_phase(layer)
        if layer < N_LAYERS:
            plsc.subcore_barrier()


_lightgcn_sc = pl.kernel(
    _body,
    out_type=(
        jax.ShapeDtypeStruct((NC * NP, HALF), _F32),
        jax.ShapeDtypeStruct((NC * NP, HALF), _F32),
    ),
    mesh=plsc.VectorSubcoreMesh(core_axis_name="c", subcore_axis_name="s"),
    compiler_params=pltpu.CompilerParams(use_tc_tiling_on_sc=False),
    scratch_types=[
        pltpu.VMEM_SHARED((NP, HALF), _F32),
        pltpu.VMEM_SHARED((NP, HALF), _F32),
        pltpu.VMEM_SHARED((NP,), _F32),
        pltpu.VMEM_SHARED((NP,), _F32),
        pltpu.VMEM((2, G, EB), _I32),
        pltpu.VMEM((2, G, EB), _I32),
        pltpu.VMEM((2, EB, HALF), _F32),
        pltpu.VMEM((NB, HALF), _F32),
        pltpu.VMEM((NB, HALF), _F32),
        pltpu.VMEM((2, TN), _F32),
        pltpu.VMEM((EB,), _F32),
        pltpu.VMEM((TN,), _F32),
    ] + [pltpu.SemaphoreType.DMA] * 9,
)


@jax.jit
def kernel(edge_index, embedding):
    src = edge_index[0].astype(_I32)
    dst = edge_index[1].astype(_I32)
    pad_e = EPAD - N_EDGES
    fill = jnp.full((pad_e,), N_NODES, _I32)
    srcp = jnp.concatenate([src, fill]).reshape(NS, CHUNKS, EB)
    dstp = jnp.concatenate([dst, fill]).reshape(NS, CHUNKS, EB)
    embp = jnp.pad(embedding, ((0, NP - N_NODES), (0, 0)))
    emb_r = embp.reshape(NP, NC, HALF).transpose(1, 0, 2).reshape(NC * NP, HALF)

    out_r, h_r = _lightgcn_sc(srcp, dstp, emb_r)

    def _unsplit(a):
        return (a.reshape(NC, NP, HALF).transpose(1, 0, 2)
                .reshape(NP, DIM)[:N_NODES])

    return (_unsplit(out_r), _unsplit(h_r))
